# Initial kernel scaffold; baseline (speedup 1.0000x reference)
import jax
import jax.numpy as jnp
from jax.experimental import pallas as pl


def kernel(h, edge_index, d, gate_W, gate_b):
    raise NotImplementedError("write your pallas kernel here")


# speedup vs baseline: 18.1174x; 18.1174x over previous
import jax
import jax.numpy as jnp
from jax import lax
from jax.experimental import pallas as pl
from jax.experimental.pallas import tpu as pltpu
from jax.experimental.pallas import tpu_sc as plsc

N_NODES = 10000
N_EDGES = 320000
D_FEAT = 128

_INFO = plsc.get_sparse_core_info()
NC = _INFO.num_cores
NS = _INFO.num_subcores
LANES = _INFO.num_lanes
D_HALF = D_FEAT // NC
EDGES_PER_T = N_EDGES // NS
BATCH = 800
NBATCH = EDGES_PER_T // BATCH
ROWS_PER_TILE = 624
TAIL_ROWS = N_NODES - NS * ROWS_PER_TILE


def _proj_body(h_ref, w_ref, bias_ref, out_ref):
    out_ref[...] = (
        jnp.dot(h_ref[...], w_ref[...], preferred_element_type=jnp.float32)
        + bias_ref[...]
    )


def _proj(h, wt, bias2):
    return pl.pallas_call(
        _proj_body,
        out_shape=jax.ShapeDtypeStruct((N_NODES, 2), jnp.float32),
    )(h, wt, bias2)


def _edges_body(src_hbm, dst_hbm, a_hbm, b_hbm, d_hbm, h0_hbm, h1_hbm,
                out_hbm,
                a_v, b_v, d_v, src_v, dst_v, e_v, rows_v, z_sh, sem):
    cid = lax.axis_index("c")
    sid = lax.axis_index("s")
    base = sid * EDGES_PER_T

    pltpu.sync_copy(a_hbm, a_v)
    pltpu.sync_copy(b_hbm, b_v)
    pltpu.sync_copy(d_hbm, d_v)

    def _zero_row(j, _):
        for k in range(D_HALF // LANES):
            rows_v[j, pl.ds(k * LANES, LANES)] = jnp.zeros(
                (LANES,), jnp.float32)
        return 0
    lax.fori_loop(0, ROWS_PER_TILE, _zero_row, 0)

    r0 = sid * ROWS_PER_TILE
    pltpu.sync_copy(rows_v.at[pl.ds(0, ROWS_PER_TILE)],
                    z_sh.at[pl.ds(r0, ROWS_PER_TILE)])

    @pl.when(sid == NS - 1)
    def _zero_tail():
        pltpu.sync_copy(rows_v.at[pl.ds(0, TAIL_ROWS)],
                        z_sh.at[pl.ds(NS * ROWS_PER_TILE, TAIL_ROWS)])

    plsc.subcore_barrier()

    zeros16 = jnp.zeros((LANES,), jnp.int32)

    def _batch(g, _):
        eb = base + g * BATCH
        pltpu.sync_copy(src_hbm.at[pl.ds(eb, BATCH)], src_v)
        pltpu.sync_copy(dst_hbm.at[pl.ds(eb, BATCH)], dst_v)

        @pl.when(cid == 0)
        def _gat0():
            pltpu.async_copy(h0_hbm.at[src_v], rows_v, sem)

        @pl.when(cid == 1)
        def _gat1():
            pltpu.async_copy(h1_hbm.at[src_v], rows_v, sem)

        def _gate(j, _):
            s16 = src_v[pl.ds(j * LANES, LANES)]
            t16 = dst_v[pl.ds(j * LANES, LANES)]
            av = plsc.load_gather(a_v, [t16])
            bv = plsc.load_gather(b_v, [s16])
            dd = plsc.load_gather(d_v, [t16])
            ds_ = plsc.load_gather(d_v, [s16])
            t = av + bv
            gt = 1.0 - 2.0 / (jnp.exp(2.0 * t) + 1.0)
            e_v[pl.ds(j * LANES, LANES)] = gt * dd * ds_
            return 0
        lax.fori_loop(0, BATCH // LANES, _gate, 0)

        pltpu.make_async_copy(h0_hbm.at[src_v], rows_v, sem).wait()

        def _scale(j, _):
            esp = plsc.load_gather(e_v, [zeros16 + j])
            for k in range(D_HALF // LANES):
                sl = pl.ds(k * LANES, LANES)
                rows_v[j, sl] = rows_v[j, sl] * esp
            return 0
        lax.fori_loop(0, BATCH, _scale, 0)

        pltpu.sync_copy(rows_v.at[pl.ds(0, BATCH)], z_sh.at[dst_v], add=True)
        return 0

    lax.fori_loop(0, NBATCH, _batch, 0)
    plsc.subcore_barrier()

    pltpu.sync_copy(z_sh.at[pl.ds(r0, ROWS_PER_TILE)],
                    out_hbm.at[cid, pl.ds(r0, ROWS_PER_TILE)])

    @pl.when(sid == NS - 1)
    def _copy_tail():
        pltpu.sync_copy(z_sh.at[pl.ds(NS * ROWS_PER_TILE, TAIL_ROWS)],
                        out_hbm.at[cid, pl.ds(NS * ROWS_PER_TILE, TAIL_ROWS)])


@jax.jit
def _run(h, edge_index, d, proj):
    src = edge_index[0]
    dst = edge_index[1]
    a = proj[:, 0]
    b = proj[:, 1]
    h0 = h[:, :D_HALF]
    h1 = h[:, D_HALF:]
    mesh = plsc.VectorSubcoreMesh(core_axis_name="c", subcore_axis_name="s")
    out = pl.kernel(
        _edges_body,
        out_type=jax.ShapeDtypeStruct((NC, N_NODES, D_HALF), jnp.float32),
        mesh=mesh,
        compiler_params=pltpu.CompilerParams(
            needs_layout_passes=False, use_tc_tiling_on_sc=False),
        scratch_types=[
            pltpu.VMEM((N_NODES,), jnp.float32),
            pltpu.VMEM((N_NODES,), jnp.float32),
            pltpu.VMEM((N_NODES,), jnp.float32),
            pltpu.VMEM((BATCH,), jnp.int32),
            pltpu.VMEM((BATCH,), jnp.int32),
            pltpu.VMEM((BATCH,), jnp.float32),
            pltpu.VMEM((BATCH, D_HALF), jnp.float32),
            pltpu.VMEM_SHARED((N_NODES, D_HALF), jnp.float32),
            pltpu.SemaphoreType.DMA,
        ],
    )(src, dst, a, b, d, h0, h1)
    return jnp.concatenate([out[0], out[1]], axis=1)


def kernel(h, edge_index, d, gate_W, gate_b):
    w_dst = gate_W[0, :D_FEAT]
    w_src = gate_W[0, D_FEAT:]
    wt = jnp.stack([w_dst, w_src], axis=1)
    bias2 = jnp.stack([gate_b[0], jnp.zeros((), jnp.float32)]).reshape(1, 2)
    proj = _proj(h, wt, bias2)
    return _run(h, edge_index, d, proj)
